# Initial kernel scaffold; baseline (speedup 1.0000x reference)
import jax
import jax.numpy as jnp
from jax.experimental import pallas as pl


def kernel(x, batch, ps, qs, W, b):
    raise NotImplementedError("write your pallas kernel here")


# speedup vs baseline: 1.8768x; 1.8768x over previous
import functools

import jax
import jax.numpy as jnp
from jax import lax
from jax.experimental import pallas as pl
from jax.experimental.pallas import tpu as pltpu
from jax.experimental.pallas import tpu_sc as plsc

N = 320000
D = 128
B = 1024
EPS = 1e-6

NC = 2
NS = 16
NW = NC * NS
ROWS_PER = N // NW
CHUNK = 400
NCHUNK = ROWS_PER // CHUNK
WIN = 128
AC = D + 16


def _lane(v, i):
    return jnp.sum(jnp.where(lax.iota(jnp.int32, 16) == i, v, jnp.zeros_like(v)))


_P1_BLOCK = 2000


def _pow_body(p_ref, x_ref, y_ref):
    p = p_ref[0, 0]
    xa = jnp.abs(x_ref[...]) + EPS
    y_ref[...] = jnp.exp(p * jnp.log(xa))


def _phase1(x, p_arr):
    return pl.pallas_call(
        _pow_body,
        grid=(N // _P1_BLOCK,),
        in_specs=[
            pl.BlockSpec(memory_space=pltpu.SMEM),
            pl.BlockSpec((_P1_BLOCK, D), lambda i: (i, 0)),
        ],
        out_specs=pl.BlockSpec((_P1_BLOCK, D), lambda i: (i, 0)),
        out_shape=jax.ShapeDtypeStruct((N, D), jnp.float32),
    )(p_arr, x)


def _sc_body(y_hbm, batch_hbm, outs_hbm, outp_hbm, ybuf, bbuf, tmpv, acc):
    cid = lax.axis_index("c")
    sid = lax.axis_index("s")
    wid = sid * NC + cid
    i0 = wid * ROWS_PER

    zvec = jnp.zeros((16,), jnp.float32)
    e0 = jnp.where(lax.iota(jnp.int32, 16) == 0, 1.0, 0.0).astype(jnp.float32)

    def zero_acc():
        def zrow(r, _):
            for j in range(AC // 16):
                acc[r, pl.ds(16 * j, 16)] = zvec
            return 0
        lax.fori_loop(0, WIN, zrow, 0)

    zero_acc()

    @pl.when(wid > 0)
    def _():
        pltpu.sync_copy(batch_hbm.at[pl.ds(i0 - 16, 16)], tmpv)
    prev_raw = _lane(tmpv[...], 15)
    prev_seg = jnp.where(wid > 0, prev_raw, jnp.int32(-1))

    pltpu.sync_copy(batch_hbm.at[pl.ds(i0 + ROWS_PER - 16, 16)], tmpv)
    last_seg = _lane(tmpv[...], 15)

    idv = jnp.where(lax.iota(jnp.int32, 16) == 1,
                    prev_seg.astype(jnp.float32), 0.0)
    plsc.addupdate(acc.at[0, pl.ds(D, 16)], idv)

    def flush(base, first, new_base):
        @pl.when(first)
        def _():
            pltpu.sync_copy(acc.at[pl.ds(0, 1)], outp_hbm.at[pl.ds(wid, 1)])
            pltpu.sync_copy(acc.at[pl.ds(1, WIN - 1)],
                            outs_hbm.at[pl.ds(base + 1, WIN - 1)])

        @pl.when(jnp.logical_not(first))
        def _():
            pltpu.sync_copy(acc.at[pl.ds(0, WIN)],
                            outs_hbm.at[pl.ds(base, WIN)])

        zero_acc()

        def zfill(k, _):
            pltpu.sync_copy(acc.at[pl.ds(0, 1)],
                            outs_hbm.at[pl.ds(base + WIN + k, 1)])
            return 0
        lax.fori_loop(0, new_base - (base + WIN), zfill, 0)

    def chunk_body(ci, carry):
        row0 = i0 + ci * CHUNK
        pltpu.sync_copy(y_hbm.at[pl.ds(row0, CHUNK)], ybuf)
        pltpu.sync_copy(batch_hbm.at[pl.ds(row0, CHUNK)], bbuf)

        def group_body(g, carry):
            segv = bbuf[pl.ds(g * 16, 16)]

            def row_body(r, carry):
                base, first = carry
                sseg = _lane(segv, r)

                def do_flush(_):
                    flush(base, first, sseg)
                    return sseg, jnp.bool_(False)

                base, first = lax.cond(sseg - base >= WIN, do_flush,
                                       lambda _: (base, first), None)
                slocal = sseg - base
                rr = g * 16 + r
                for j in range(D // 16):
                    plsc.addupdate(acc.at[slocal, pl.ds(16 * j, 16)],
                                   ybuf[rr, pl.ds(16 * j, 16)])
                plsc.addupdate(acc.at[slocal, pl.ds(D, 16)], e0)
                return base, first

            return lax.fori_loop(0, 16, row_body, carry)

        return lax.fori_loop(0, CHUNK // 16, group_body, carry)

    base, first = lax.fori_loop(0, NCHUNK, chunk_body,
                                (prev_seg, jnp.bool_(True)))

    @pl.when(first)
    def _():
        pltpu.sync_copy(acc.at[pl.ds(0, 1)], outp_hbm.at[pl.ds(wid, 1)])

    start_r = jnp.where(first, 1, 0)

    def frow(k, _):
        r = start_r + k
        pltpu.sync_copy(acc.at[pl.ds(r, 1)], outs_hbm.at[pl.ds(base + r, 1)])
        return 0
    lax.fori_loop(0, last_seg - base - start_r + 1, frow, 0)

    @pl.when(wid == NW - 1)
    def _():
        zero_acc()

        def ztail(k, _):
            pltpu.sync_copy(acc.at[pl.ds(0, 1)],
                            outs_hbm.at[pl.ds(last_seg + 1 + k, 1)])
            return 0
        lax.fori_loop(0, B - 1 - last_seg, ztail, 0)


def _phase2(y, batch):
    mesh = plsc.VectorSubcoreMesh(core_axis_name="c", subcore_axis_name="s",
                                  num_cores=NC, num_subcores=NS)
    f = pl.kernel(
        _sc_body,
        out_type=(
            jax.ShapeDtypeStruct((B, AC), jnp.float32),
            jax.ShapeDtypeStruct((NW, AC), jnp.float32),
        ),
        mesh=mesh,
        compiler_params=pltpu.CompilerParams(use_tc_tiling_on_sc=False,
                                             needs_layout_passes=False),
        scratch_types=[
            pltpu.VMEM((CHUNK, D), jnp.float32),
            pltpu.VMEM((CHUNK,), jnp.int32),
            pltpu.VMEM((16,), jnp.int32),
            pltpu.VMEM((WIN, AC), jnp.float32),
        ],
    )
    return f(y, batch)


def _fin_body(sc_ref, s_ref, p_ref, w_ref, b_ref, o_ref):
    pinv = sc_ref[0, 0]
    q = sc_ref[0, 1]
    s = s_ref[...]
    part = p_ref[...]
    ids = part[:, D + 1].astype(jnp.int32)
    rows = lax.broadcasted_iota(jnp.int32, (B, NW), 0)
    onehot = (rows == ids[None, :]).astype(jnp.float32)
    s = s + jnp.dot(onehot, part, preferred_element_type=jnp.float32)
    pooled = s[:, :D]
    nums = s[:, D:D + 1]
    gnp = jnp.exp(pinv * jnp.log(pooled)) * jnp.exp(-q * jnp.log(nums))
    o_ref[...] = (
        jax.lax.dot_general(gnp, w_ref[...], (((1,), (1,)), ((), ())),
                            preferred_element_type=jnp.float32)
        + b_ref[...][None, :]
    )


def _phase3(scal, s, partials, W, b):
    return pl.pallas_call(
        _fin_body,
        in_specs=[
            pl.BlockSpec(memory_space=pltpu.SMEM),
            pl.BlockSpec(memory_space=pltpu.VMEM),
            pl.BlockSpec(memory_space=pltpu.VMEM),
            pl.BlockSpec(memory_space=pltpu.VMEM),
            pl.BlockSpec(memory_space=pltpu.VMEM),
        ],
        out_shape=jax.ShapeDtypeStruct((B, D), jnp.float32),
    )(scal, s, partials, W, b)


def kernel(x, batch, ps, qs, W, b):
    neg_sp = -jax.nn.softplus(ps)
    thres = jnp.where(neg_sp > -50.0, neg_sp, 50.0)
    p = -thres[0]
    q = jnp.tanh(qs)[0]
    batch = batch.astype(jnp.int32)

    y = _phase1(x, p.reshape(1, 1))
    s, partials = _phase2(y, batch)
    scal = jnp.stack([1.0 / p, q]).reshape(1, 2)
    return _phase3(scal, s, partials, W, b)


# speedup vs baseline: 2.5721x; 1.3705x over previous
import functools

import jax
import jax.numpy as jnp
from jax import lax
from jax.experimental import pallas as pl
from jax.experimental.pallas import tpu as pltpu
from jax.experimental.pallas import tpu_sc as plsc

N = 320000
D = 128
B = 1024
EPS = 1e-6

NC = 2
NS = 16
NW = NC * NS
ROWS_PER = N // NW
CHUNK = 400
NCHUNK = ROWS_PER // CHUNK
WIN = 128
AC = D + 16


def _lane(v, i):
    return jnp.sum(jnp.where(lax.iota(jnp.int32, 16) == i, v, jnp.zeros_like(v)))


_P1_BLOCK = 2000


def _pow_body(p_ref, x_ref, y_ref):
    p = p_ref[0, 0]
    xa = jnp.abs(x_ref[...]) + EPS
    y_ref[...] = jnp.exp(p * jnp.log(xa))


def _phase1(x, p_arr):
    return pl.pallas_call(
        _pow_body,
        grid=(N // _P1_BLOCK,),
        in_specs=[
            pl.BlockSpec(memory_space=pltpu.SMEM),
            pl.BlockSpec((_P1_BLOCK, D), lambda i: (i, 0)),
        ],
        out_specs=pl.BlockSpec((_P1_BLOCK, D), lambda i: (i, 0)),
        out_shape=jax.ShapeDtypeStruct((N, D), jnp.float32),
    )(p_arr, x)


def _sc_body(y_hbm, batch_hbm, outs_hbm, outp_hbm, ybuf, bbuf, tmpv, acc):
    cid = lax.axis_index("c")
    sid = lax.axis_index("s")
    wid = sid * NC + cid
    i0 = wid * ROWS_PER

    zvec = jnp.zeros((16,), jnp.float32)
    e0 = jnp.where(lax.iota(jnp.int32, 16) == 0, 1.0, 0.0).astype(jnp.float32)

    def zero_acc():
        def zrow(r, _):
            for j in range(AC // 16):
                acc[r, pl.ds(16 * j, 16)] = zvec
            return 0
        lax.fori_loop(0, WIN, zrow, 0)

    zero_acc()

    @pl.when(wid > 0)
    def _():
        pltpu.sync_copy(batch_hbm.at[pl.ds(i0 - 16, 16)], tmpv)
    prev_raw = _lane(tmpv[...], 15)
    prev_seg = jnp.where(wid > 0, prev_raw, jnp.int32(-1))

    pltpu.sync_copy(batch_hbm.at[pl.ds(i0 + ROWS_PER - 16, 16)], tmpv)
    last_seg = _lane(tmpv[...], 15)

    idv = jnp.where(lax.iota(jnp.int32, 16) == 1,
                    prev_seg.astype(jnp.float32), 0.0)
    plsc.addupdate(acc.at[0, pl.ds(D, 16)], idv)

    def flush(base, first, new_base):
        @pl.when(first)
        def _():
            pltpu.sync_copy(acc.at[pl.ds(0, 1)], outp_hbm.at[pl.ds(wid, 1)])
            pltpu.sync_copy(acc.at[pl.ds(1, WIN - 1)],
                            outs_hbm.at[pl.ds(base + 1, WIN - 1)])

        @pl.when(jnp.logical_not(first))
        def _():
            pltpu.sync_copy(acc.at[pl.ds(0, WIN)],
                            outs_hbm.at[pl.ds(base, WIN)])

        zero_acc()

        def zfill(k, _):
            pltpu.sync_copy(acc.at[pl.ds(0, 1)],
                            outs_hbm.at[pl.ds(base + WIN + k, 1)])
            return 0
        lax.fori_loop(0, new_base - (base + WIN), zfill, 0)

    def accum_row(rr, slocal):
        for j in range(D // 16):
            plsc.addupdate(acc.at[slocal, pl.ds(16 * j, 16)],
                           ybuf[rr, pl.ds(16 * j, 16)])
        plsc.addupdate(acc.at[slocal, pl.ds(D, 16)], e0)

    def chunk_body(ci, carry):
        row0 = i0 + ci * CHUNK
        pltpu.sync_copy(y_hbm.at[pl.ds(row0, CHUNK)], ybuf)
        pltpu.sync_copy(batch_hbm.at[pl.ds(row0, CHUNK)], bbuf)
        base_in, first_in = carry
        chunk_last = _lane(bbuf[pl.ds(CHUNK - 16, 16)], 15)

        def fast_path(carry):
            base, first = carry

            def group_body(g, _):
                segv = bbuf[pl.ds(g * 16, 16)]
                slv = segv - base
                for r in range(16):
                    accum_row(g * 16 + r, _lane(slv, r))
                return 0

            lax.fori_loop(0, CHUNK // 16, group_body, 0)
            return base, first

        def slow_path(carry):
            def row_body(r, carry):
                base, first = carry
                sseg = _lane(bbuf[pl.ds((r // 16) * 16, 16)], r % 16)

                def do_flush(_):
                    flush(base, first, sseg)
                    return sseg, jnp.bool_(False)

                base, first = lax.cond(sseg - base >= WIN, do_flush,
                                       lambda _: (base, first), None)
                accum_row(r, sseg - base)
                return base, first

            return lax.fori_loop(0, CHUNK, row_body, carry)

        return lax.cond(chunk_last - base_in < WIN, fast_path, slow_path,
                        (base_in, first_in))

    base, first = lax.fori_loop(0, NCHUNK, chunk_body,
                                (prev_seg, jnp.bool_(True)))

    @pl.when(first)
    def _():
        pltpu.sync_copy(acc.at[pl.ds(0, 1)], outp_hbm.at[pl.ds(wid, 1)])

    start_r = jnp.where(first, 1, 0)

    def frow(k, _):
        r = start_r + k
        pltpu.sync_copy(acc.at[pl.ds(r, 1)], outs_hbm.at[pl.ds(base + r, 1)])
        return 0
    lax.fori_loop(0, last_seg - base - start_r + 1, frow, 0)

    @pl.when(wid == NW - 1)
    def _():
        zero_acc()

        def ztail(k, _):
            pltpu.sync_copy(acc.at[pl.ds(0, 1)],
                            outs_hbm.at[pl.ds(last_seg + 1 + k, 1)])
            return 0
        lax.fori_loop(0, B - 1 - last_seg, ztail, 0)


def _phase2(y, batch):
    mesh = plsc.VectorSubcoreMesh(core_axis_name="c", subcore_axis_name="s",
                                  num_cores=NC, num_subcores=NS)
    f = pl.kernel(
        _sc_body,
        out_type=(
            jax.ShapeDtypeStruct((B, AC), jnp.float32),
            jax.ShapeDtypeStruct((NW, AC), jnp.float32),
        ),
        mesh=mesh,
        compiler_params=pltpu.CompilerParams(use_tc_tiling_on_sc=False,
                                             needs_layout_passes=False),
        scratch_types=[
            pltpu.VMEM((CHUNK, D), jnp.float32),
            pltpu.VMEM((CHUNK,), jnp.int32),
            pltpu.VMEM((16,), jnp.int32),
            pltpu.VMEM((WIN, AC), jnp.float32),
        ],
    )
    return f(y, batch)


def _fin_body(sc_ref, s_ref, p_ref, w_ref, b_ref, o_ref):
    pinv = sc_ref[0, 0]
    q = sc_ref[0, 1]
    s = s_ref[...]
    part = p_ref[...]
    ids = part[:, D + 1].astype(jnp.int32)
    rows = lax.broadcasted_iota(jnp.int32, (B, NW), 0)
    onehot = (rows == ids[None, :]).astype(jnp.float32)
    s = s + jnp.dot(onehot, part, preferred_element_type=jnp.float32)
    pooled = s[:, :D]
    nums = s[:, D:D + 1]
    gnp = jnp.exp(pinv * jnp.log(pooled)) * jnp.exp(-q * jnp.log(nums))
    o_ref[...] = (
        jax.lax.dot_general(gnp, w_ref[...], (((1,), (1,)), ((), ())),
                            preferred_element_type=jnp.float32)
        + b_ref[...][None, :]
    )


def _phase3(scal, s, partials, W, b):
    return pl.pallas_call(
        _fin_body,
        in_specs=[
            pl.BlockSpec(memory_space=pltpu.SMEM),
            pl.BlockSpec(memory_space=pltpu.VMEM),
            pl.BlockSpec(memory_space=pltpu.VMEM),
            pl.BlockSpec(memory_space=pltpu.VMEM),
            pl.BlockSpec(memory_space=pltpu.VMEM),
        ],
        out_shape=jax.ShapeDtypeStruct((B, D), jnp.float32),
    )(scal, s, partials, W, b)


def kernel(x, batch, ps, qs, W, b):
    neg_sp = -jax.nn.softplus(ps)
    thres = jnp.where(neg_sp > -50.0, neg_sp, 50.0)
    p = -thres[0]
    q = jnp.tanh(qs)[0]
    batch = batch.astype(jnp.int32)

    y = _phase1(x, p.reshape(1, 1))
    s, partials = _phase2(y, batch)
    scal = jnp.stack([1.0 / p, q]).reshape(1, 2)
    return _phase3(scal, s, partials, W, b)


# speedup vs baseline: 2.5925x; 1.0079x over previous
import functools

import jax
import jax.numpy as jnp
from jax import lax
from jax.experimental import pallas as pl
from jax.experimental.pallas import tpu as pltpu
from jax.experimental.pallas import tpu_sc as plsc

N = 320000
D = 128
B = 1024
EPS = 1e-6

NC = 2
NS = 16
NW = NC * NS
ROWS_PER = N // NW
CHUNK = 400
NCHUNK = ROWS_PER // CHUNK
WIN = 128
AC = D + 32


def _lane(v, i):
    return jnp.sum(jnp.where(lax.iota(jnp.int32, 16) == i, v, jnp.zeros_like(v)))


_BCAST_DN = lax.GatherDimensionNumbers(
    offset_dims=(), collapsed_slice_dims=(0,), start_index_map=(0,))


def _bcast_lane(v, r):
    idx = jnp.full((16, 1), r, jnp.int32)
    return lax.gather(v, idx, _BCAST_DN, (1,),
                      mode=lax.GatherScatterMode.PROMISE_IN_BOUNDS)


_P1_BLOCK = 2000


def _pow_body(p_ref, x_ref, y_ref):
    p = p_ref[0, 0]
    xa = jnp.abs(x_ref[...]) + EPS
    y_ref[...] = jnp.exp(p * jnp.log(xa))


def _phase1(x, p_arr):
    return pl.pallas_call(
        _pow_body,
        grid=(N // _P1_BLOCK,),
        in_specs=[
            pl.BlockSpec(memory_space=pltpu.SMEM),
            pl.BlockSpec((_P1_BLOCK, D), lambda i: (i, 0)),
        ],
        out_specs=pl.BlockSpec((_P1_BLOCK, D), lambda i: (i, 0)),
        out_shape=jax.ShapeDtypeStruct((N, D), jnp.float32),
    )(p_arr, x)


def _sc_body(y_hbm, batch_hbm, outs_hbm, outp_hbm, ybuf, bbuf, tmpv, acc):
    cid = lax.axis_index("c")
    sid = lax.axis_index("s")
    wid = sid * NC + cid
    i0 = wid * ROWS_PER

    zvec = jnp.zeros((16,), jnp.float32)
    e0 = jnp.where(lax.iota(jnp.int32, 16) == 0, 1.0, 0.0).astype(jnp.float32)

    def zero_acc():
        def zrow(r, _):
            for j in range(AC // 16):
                acc[r, pl.ds(16 * j, 16)] = zvec
            return 0
        lax.fori_loop(0, WIN, zrow, 0)

    zero_acc()

    @pl.when(wid > 0)
    def _():
        pltpu.sync_copy(batch_hbm.at[pl.ds(i0 - 16, 16)], tmpv)
    prev_raw = _lane(tmpv[...], 15)
    prev_seg = jnp.where(wid > 0, prev_raw, jnp.int32(-1))

    pltpu.sync_copy(batch_hbm.at[pl.ds(i0 + ROWS_PER - 16, 16)], tmpv)
    last_seg = _lane(tmpv[...], 15)

    idv = jnp.where(lax.iota(jnp.int32, 16) == 0,
                    prev_seg.astype(jnp.float32), 0.0)
    plsc.addupdate(acc.at[0, pl.ds(AC - 16, 16)], idv)

    def flush(base, first, new_base):
        @pl.when(first)
        def _():
            pltpu.sync_copy(acc.at[pl.ds(0, 1)], outp_hbm.at[pl.ds(wid, 1)])
            pltpu.sync_copy(acc.at[pl.ds(1, WIN - 1)],
                            outs_hbm.at[pl.ds(base + 1, WIN - 1)])

        @pl.when(jnp.logical_not(first))
        def _():
            pltpu.sync_copy(acc.at[pl.ds(0, WIN)],
                            outs_hbm.at[pl.ds(base, WIN)])

        zero_acc()

        def zfill(k, _):
            pltpu.sync_copy(acc.at[pl.ds(0, 1)],
                            outs_hbm.at[pl.ds(base + WIN + k, 1)])
            return 0
        lax.fori_loop(0, new_base - (base + WIN), zfill, 0)

    def accum_row(rr, slocal):
        for j in range(D // 16):
            plsc.addupdate(acc.at[slocal, pl.ds(16 * j, 16)],
                           ybuf[rr, pl.ds(16 * j, 16)])
        plsc.addupdate(acc.at[slocal, pl.ds(D, 16)], e0)

    def chunk_body(ci, carry):
        row0 = i0 + ci * CHUNK
        pltpu.sync_copy(y_hbm.at[pl.ds(row0, CHUNK)], ybuf)
        pltpu.sync_copy(batch_hbm.at[pl.ds(row0, CHUNK)], bbuf)
        base_in, first_in = carry
        chunk_last = _lane(bbuf[pl.ds(CHUNK - 16, 16)], 15)

        def fast_path(carry):
            base, first = carry

            lanes = lax.iota(jnp.int32, 16)
            ones = jnp.ones((16,), jnp.float32)

            def group_body(g, _):
                segv = bbuf[pl.ds(g * 16, 16)]
                slv = segv - base
                plsc.addupdate_scatter(acc, [slv, D + lanes], ones)
                for r in range(16):
                    rowv = _bcast_lane(slv, r)
                    rr = g * 16 + r
                    for j in range(D // 16):
                        plsc.addupdate_scatter(
                            acc, [rowv, 16 * j + lanes],
                            ybuf[rr, pl.ds(16 * j, 16)])
                return 0

            lax.fori_loop(0, CHUNK // 16, group_body, 0)
            return base, first

        def slow_path(carry):
            def row_body(r, carry):
                base, first = carry
                sseg = _lane(bbuf[pl.ds((r // 16) * 16, 16)], r % 16)

                def do_flush(_):
                    flush(base, first, sseg)
                    return sseg, jnp.bool_(False)

                base, first = lax.cond(sseg - base >= WIN, do_flush,
                                       lambda _: (base, first), None)
                accum_row(r, sseg - base)
                return base, first

            return lax.fori_loop(0, CHUNK, row_body, carry)

        return lax.cond(chunk_last - base_in < WIN, fast_path, slow_path,
                        (base_in, first_in))

    base, first = lax.fori_loop(0, NCHUNK, chunk_body,
                                (prev_seg, jnp.bool_(True)))

    @pl.when(first)
    def _():
        pltpu.sync_copy(acc.at[pl.ds(0, 1)], outp_hbm.at[pl.ds(wid, 1)])

    start_r = jnp.where(first, 1, 0)

    def frow(k, _):
        r = start_r + k
        pltpu.sync_copy(acc.at[pl.ds(r, 1)], outs_hbm.at[pl.ds(base + r, 1)])
        return 0
    lax.fori_loop(0, last_seg - base - start_r + 1, frow, 0)

    @pl.when(wid == NW - 1)
    def _():
        zero_acc()

        def ztail(k, _):
            pltpu.sync_copy(acc.at[pl.ds(0, 1)],
                            outs_hbm.at[pl.ds(last_seg + 1 + k, 1)])
            return 0
        lax.fori_loop(0, B - 1 - last_seg, ztail, 0)


def _phase2(y, batch):
    mesh = plsc.VectorSubcoreMesh(core_axis_name="c", subcore_axis_name="s",
                                  num_cores=NC, num_subcores=NS)
    f = pl.kernel(
        _sc_body,
        out_type=(
            jax.ShapeDtypeStruct((B, AC), jnp.float32),
            jax.ShapeDtypeStruct((NW, AC), jnp.float32),
        ),
        mesh=mesh,
        compiler_params=pltpu.CompilerParams(use_tc_tiling_on_sc=False,
                                             needs_layout_passes=False),
        scratch_types=[
            pltpu.VMEM((CHUNK, D), jnp.float32),
            pltpu.VMEM((CHUNK,), jnp.int32),
            pltpu.VMEM((16,), jnp.int32),
            pltpu.VMEM((WIN, AC), jnp.float32),
        ],
    )
    return f(y, batch)


def _fin_body(sc_ref, s_ref, p_ref, w_ref, b_ref, o_ref):
    pinv = sc_ref[0, 0]
    q = sc_ref[0, 1]
    s = s_ref[...]
    part = p_ref[...]
    ids = part[:, AC - 16].astype(jnp.int32)
    rows = lax.broadcasted_iota(jnp.int32, (B, NW), 0)
    onehot = (rows == ids[None, :]).astype(jnp.float32)
    s = s + jnp.dot(onehot, part, preferred_element_type=jnp.float32)
    pooled = s[:, :D]
    nums = jnp.sum(s[:, D:D + 16], axis=1, keepdims=True)
    gnp = jnp.exp(pinv * jnp.log(pooled)) * jnp.exp(-q * jnp.log(nums))
    o_ref[...] = (
        jax.lax.dot_general(gnp, w_ref[...], (((1,), (1,)), ((), ())),
                            preferred_element_type=jnp.float32)
        + b_ref[...][None, :]
    )


def _phase3(scal, s, partials, W, b):
    return pl.pallas_call(
        _fin_body,
        in_specs=[
            pl.BlockSpec(memory_space=pltpu.SMEM),
            pl.BlockSpec(memory_space=pltpu.VMEM),
            pl.BlockSpec(memory_space=pltpu.VMEM),
            pl.BlockSpec(memory_space=pltpu.VMEM),
            pl.BlockSpec(memory_space=pltpu.VMEM),
        ],
        out_shape=jax.ShapeDtypeStruct((B, D), jnp.float32),
    )(scal, s, partials, W, b)


def kernel(x, batch, ps, qs, W, b):
    neg_sp = -jax.nn.softplus(ps)
    thres = jnp.where(neg_sp > -50.0, neg_sp, 50.0)
    p = -thres[0]
    q = jnp.tanh(qs)[0]
    batch = batch.astype(jnp.int32)

    y = _phase1(x, p.reshape(1, 1))
    s, partials = _phase2(y, batch)
    scal = jnp.stack([1.0 / p, q]).reshape(1, 2)
    return _phase3(scal, s, partials, W, b)


# speedup vs baseline: 4.0368x; 1.5571x over previous
import functools

import jax
import jax.numpy as jnp
from jax import lax
from jax.experimental import pallas as pl
from jax.experimental.pallas import tpu as pltpu
from jax.experimental.pallas import tpu_sc as plsc

N = 320000
D = 128
B = 1024
EPS = 1e-6

NC = 2
NS = 16
NW = NC * NS
ROWS_PER = N // NW
CHUNK = 400
NCHUNK = ROWS_PER // CHUNK
WIN = 128
AC = D + 32


def _lane(v, i):
    return jnp.sum(jnp.where(lax.iota(jnp.int32, 16) == i, v, jnp.zeros_like(v)))


_BCAST_DN = lax.GatherDimensionNumbers(
    offset_dims=(), collapsed_slice_dims=(0,), start_index_map=(0,))


def _bcast_lane(v, r):
    idx = jnp.full((16, 1), r, jnp.int32)
    return lax.gather(v, idx, _BCAST_DN, (1,),
                      mode=lax.GatherScatterMode.PROMISE_IN_BOUNDS)


_P1_BLOCK = 2000


def _pow_body(p_ref, x_ref, y_ref):
    p = p_ref[0, 0]
    xa = jnp.abs(x_ref[...]) + EPS
    y_ref[...] = jnp.exp(p * jnp.log(xa))


def _phase1(x, p_arr):
    return pl.pallas_call(
        _pow_body,
        grid=(N // _P1_BLOCK,),
        in_specs=[
            pl.BlockSpec(memory_space=pltpu.SMEM),
            pl.BlockSpec((_P1_BLOCK, D), lambda i: (i, 0)),
        ],
        out_specs=pl.BlockSpec((_P1_BLOCK, D), lambda i: (i, 0)),
        out_shape=jax.ShapeDtypeStruct((N, D), jnp.float32),
    )(p_arr, x)


def _sc_body(y_hbm, batch_hbm, outs_hbm, outp_hbm, ybuf, bbuf, tmpv, acc):
    cid = lax.axis_index("c")
    sid = lax.axis_index("s")
    wid = sid * NC + cid
    i0 = wid * ROWS_PER

    zvec = jnp.zeros((16,), jnp.float32)
    e0 = jnp.where(lax.iota(jnp.int32, 16) == 0, 1.0, 0.0).astype(jnp.float32)

    def zero_acc():
        def zrow(r, _):
            for j in range(AC // 16):
                acc[r, pl.ds(16 * j, 16)] = zvec
            return 0
        lax.fori_loop(0, WIN, zrow, 0)

    zero_acc()

    @pl.when(wid > 0)
    def _():
        pltpu.sync_copy(batch_hbm.at[pl.ds(i0 - 16, 16)], tmpv)
    prev_raw = _lane(tmpv[...], 15)
    prev_seg = jnp.where(wid > 0, prev_raw, jnp.int32(-1))

    pltpu.sync_copy(batch_hbm.at[pl.ds(i0 + ROWS_PER - 16, 16)], tmpv)
    last_seg = _lane(tmpv[...], 15)

    idv = jnp.where(lax.iota(jnp.int32, 16) == 0,
                    prev_seg.astype(jnp.float32), 0.0)
    plsc.addupdate(acc.at[0, pl.ds(AC - 16, 16)], idv)

    def flush(base, first, new_base):
        @pl.when(first)
        def _():
            pltpu.sync_copy(acc.at[pl.ds(0, 1)], outp_hbm.at[pl.ds(wid, 1)])
            pltpu.sync_copy(acc.at[pl.ds(1, WIN - 1)],
                            outs_hbm.at[pl.ds(base + 1, WIN - 1)])

        @pl.when(jnp.logical_not(first))
        def _():
            pltpu.sync_copy(acc.at[pl.ds(0, WIN)],
                            outs_hbm.at[pl.ds(base, WIN)])

        zero_acc()

        def zfill(k, _):
            pltpu.sync_copy(acc.at[pl.ds(0, 1)],
                            outs_hbm.at[pl.ds(base + WIN + k, 1)])
            return 0
        lax.fori_loop(0, new_base - (base + WIN), zfill, 0)

    def store_run(base, cur, cnt, accs):
        row = cur - base
        for j in range(D // 16):
            acc[row, pl.ds(16 * j, 16)] = accs[j]
        acc[row, pl.ds(D, 16)] = jnp.where(lax.iota(jnp.int32, 16) == 0,
                                           cnt, 0.0)

    zaccs = tuple(jnp.zeros((16,), jnp.float32) for _ in range(D // 16))

    def chunk_body(ci, carry):
        row0 = i0 + ci * CHUNK
        pltpu.sync_copy(y_hbm.at[pl.ds(row0, CHUNK)], ybuf)
        pltpu.sync_copy(batch_hbm.at[pl.ds(row0, CHUNK)], bbuf)

        def group_body(g, carry):
            base, first, cur, cnt, accs = carry
            segv = bbuf[pl.ds(g * 16, 16)]
            allsame = jnp.all(segv == cur)

            def fast(c):
                base, first, cur, cnt, accs = c
                for r in range(16):
                    rr = g * 16 + r
                    accs = tuple(accs[j] + ybuf[rr, pl.ds(16 * j, 16)]
                                 for j in range(D // 16))
                return base, first, cur, cnt + 16.0, accs

            def slow(c):
                def row_body(r, c2):
                    base, first, cur, cnt, accs = c2
                    sseg = _lane(segv, r)
                    rr = g * 16 + r
                    yrow = tuple(ybuf[rr, pl.ds(16 * j, 16)]
                                 for j in range(D // 16))

                    def same(c3):
                        base, first, cur, cnt, accs = c3
                        na = tuple(accs[j] + yrow[j]
                                   for j in range(D // 16))
                        return base, first, cur, cnt + 1.0, na

                    def boundary(c3):
                        base, first, cur, cnt, accs = c3
                        store_run(base, cur, cnt, accs)

                        def do_flush(_):
                            flush(base, first, sseg)
                            return sseg, jnp.bool_(False)

                        base, first = lax.cond(sseg - base >= WIN, do_flush,
                                               lambda _: (base, first), None)
                        return base, first, sseg, 1.0, yrow

                    return lax.cond(sseg == cur, same, boundary, c2)

                return lax.fori_loop(0, 16, row_body, c)

            return lax.cond(allsame, fast, slow, carry)

        return lax.fori_loop(0, CHUNK // 16, group_body, carry)

    init = (prev_seg, jnp.bool_(True), prev_seg, jnp.float32(0.0), zaccs)
    base, first, cur, cnt, accs = lax.fori_loop(0, NCHUNK, chunk_body, init)
    store_run(base, cur, cnt, accs)

    @pl.when(first)
    def _():
        pltpu.sync_copy(acc.at[pl.ds(0, 1)], outp_hbm.at[pl.ds(wid, 1)])

    start_r = jnp.where(first, 1, 0)

    def frow(k, _):
        r = start_r + k
        pltpu.sync_copy(acc.at[pl.ds(r, 1)], outs_hbm.at[pl.ds(base + r, 1)])
        return 0
    lax.fori_loop(0, last_seg - base - start_r + 1, frow, 0)

    @pl.when(wid == NW - 1)
    def _():
        zero_acc()

        def ztail(k, _):
            pltpu.sync_copy(acc.at[pl.ds(0, 1)],
                            outs_hbm.at[pl.ds(last_seg + 1 + k, 1)])
            return 0
        lax.fori_loop(0, B - 1 - last_seg, ztail, 0)


def _phase2(y, batch):
    mesh = plsc.VectorSubcoreMesh(core_axis_name="c", subcore_axis_name="s",
                                  num_cores=NC, num_subcores=NS)
    f = pl.kernel(
        _sc_body,
        out_type=(
            jax.ShapeDtypeStruct((B, AC), jnp.float32),
            jax.ShapeDtypeStruct((NW, AC), jnp.float32),
        ),
        mesh=mesh,
        compiler_params=pltpu.CompilerParams(use_tc_tiling_on_sc=False,
                                             needs_layout_passes=False),
        scratch_types=[
            pltpu.VMEM((CHUNK, D), jnp.float32),
            pltpu.VMEM((CHUNK,), jnp.int32),
            pltpu.VMEM((16,), jnp.int32),
            pltpu.VMEM((WIN, AC), jnp.float32),
        ],
    )
    return f(y, batch)


def _fin_body(sc_ref, s_ref, p_ref, w_ref, b_ref, o_ref):
    pinv = sc_ref[0, 0]
    q = sc_ref[0, 1]
    s = s_ref[...]
    part = p_ref[...]
    ids = part[:, AC - 16].astype(jnp.int32)
    rows = lax.broadcasted_iota(jnp.int32, (B, NW), 0)
    onehot = (rows == ids[None, :]).astype(jnp.float32)
    s = s + jnp.dot(onehot, part, preferred_element_type=jnp.float32)
    pooled = s[:, :D]
    nums = jnp.sum(s[:, D:D + 16], axis=1, keepdims=True)
    gnp = jnp.exp(pinv * jnp.log(pooled)) * jnp.exp(-q * jnp.log(nums))
    o_ref[...] = (
        jax.lax.dot_general(gnp, w_ref[...], (((1,), (1,)), ((), ())),
                            preferred_element_type=jnp.float32)
        + b_ref[...][None, :]
    )


def _phase3(scal, s, partials, W, b):
    return pl.pallas_call(
        _fin_body,
        in_specs=[
            pl.BlockSpec(memory_space=pltpu.SMEM),
            pl.BlockSpec(memory_space=pltpu.VMEM),
            pl.BlockSpec(memory_space=pltpu.VMEM),
            pl.BlockSpec(memory_space=pltpu.VMEM),
            pl.BlockSpec(memory_space=pltpu.VMEM),
        ],
        out_shape=jax.ShapeDtypeStruct((B, D), jnp.float32),
    )(scal, s, partials, W, b)


def kernel(x, batch, ps, qs, W, b):
    neg_sp = -jax.nn.softplus(ps)
    thres = jnp.where(neg_sp > -50.0, neg_sp, 50.0)
    p = -thres[0]
    q = jnp.tanh(qs)[0]
    batch = batch.astype(jnp.int32)

    y = _phase1(x, p.reshape(1, 1))
    s, partials = _phase2(y, batch)
    scal = jnp.stack([1.0 / p, q]).reshape(1, 2)
    return _phase3(scal, s, partials, W, b)


# speedup vs baseline: 5.9176x; 1.4659x over previous
import functools

import jax
import jax.numpy as jnp
from jax import lax
from jax.experimental import pallas as pl
from jax.experimental.pallas import tpu as pltpu
from jax.experimental.pallas import tpu_sc as plsc

N = 320000
D = 128
B = 1024
EPS = 1e-6

NC = 2
NS = 16
NW = NC * NS
ROWS_PER = N // NW
CHUNK = 400
NCHUNK = ROWS_PER // CHUNK
WIN = 128
AC = D + 32


def _lane(v, i):
    return jnp.sum(jnp.where(lax.iota(jnp.int32, 16) == i, v, jnp.zeros_like(v)))


_BCAST_DN = lax.GatherDimensionNumbers(
    offset_dims=(), collapsed_slice_dims=(0,), start_index_map=(0,))


def _bcast_lane(v, r):
    idx = jnp.full((16, 1), r, jnp.int32)
    return lax.gather(v, idx, _BCAST_DN, (1,),
                      mode=lax.GatherScatterMode.PROMISE_IN_BOUNDS)


_P1_BLOCK = 2000


def _pow_body(p_ref, x_ref, y_ref):
    p = p_ref[0, 0]
    xa = jnp.abs(x_ref[...]) + EPS
    yb = jnp.exp(p * jnp.log(xa)).astype(jnp.bfloat16)
    y_ref[...] = pltpu.bitcast(yb, jnp.int32)


def _phase1(x, p_arr):
    return pl.pallas_call(
        _pow_body,
        grid=(N // _P1_BLOCK,),
        in_specs=[
            pl.BlockSpec(memory_space=pltpu.SMEM),
            pl.BlockSpec((_P1_BLOCK, D), lambda i: (i, 0)),
        ],
        out_specs=pl.BlockSpec((_P1_BLOCK // 2, D), lambda i: (i, 0)),
        out_shape=jax.ShapeDtypeStruct((N // 2, D), jnp.int32),
    )(p_arr, x)


def _sc_body(y_hbm, batch_hbm, outs_hbm, outp_hbm, ybuf2, bbuf2, tmpv, acc,
             sem0, sem1):
    cid = lax.axis_index("c")
    sid = lax.axis_index("s")
    wid = sid * NC + cid
    i0 = wid * ROWS_PER

    zvec = jnp.zeros((16,), jnp.float32)
    e0 = jnp.where(lax.iota(jnp.int32, 16) == 0, 1.0, 0.0).astype(jnp.float32)

    def zero_acc():
        def zrow(r, _):
            for j in range(AC // 16):
                acc[r, pl.ds(16 * j, 16)] = zvec
            return 0
        lax.fori_loop(0, WIN, zrow, 0)

    zero_acc()

    @pl.when(wid > 0)
    def _():
        pltpu.sync_copy(batch_hbm.at[pl.ds(i0 - 16, 16)], tmpv)
    prev_raw = _lane(tmpv[...], 15)
    prev_seg = jnp.where(wid > 0, prev_raw, jnp.int32(-1))

    pltpu.sync_copy(batch_hbm.at[pl.ds(i0 + ROWS_PER - 16, 16)], tmpv)
    last_seg = _lane(tmpv[...], 15)

    idv = jnp.where(lax.iota(jnp.int32, 16) == 0,
                    prev_seg.astype(jnp.float32), 0.0)
    plsc.addupdate(acc.at[0, pl.ds(AC - 16, 16)], idv)

    def flush(base, first, new_base):
        @pl.when(first)
        def _():
            pltpu.sync_copy(acc.at[pl.ds(0, 1)], outp_hbm.at[pl.ds(wid, 1)])
            pltpu.sync_copy(acc.at[pl.ds(1, WIN - 1)],
                            outs_hbm.at[pl.ds(base + 1, WIN - 1)])

        @pl.when(jnp.logical_not(first))
        def _():
            pltpu.sync_copy(acc.at[pl.ds(0, WIN)],
                            outs_hbm.at[pl.ds(base, WIN)])

        zero_acc()

        def zfill(k, _):
            pltpu.sync_copy(acc.at[pl.ds(0, 1)],
                            outs_hbm.at[pl.ds(base + WIN + k, 1)])
            return 0
        lax.fori_loop(0, new_base - (base + WIN), zfill, 0)

    def store_run(base, cur, cnt, accs):
        row = cur - base
        for j in range(D // 16):
            acc[row, pl.ds(16 * j, 16)] = accs[j]
        acc[row, pl.ds(D, 16)] = jnp.where(lax.iota(jnp.int32, 16) == 0,
                                           cnt, 0.0)

    zaccs = tuple(jnp.zeros((16,), jnp.float32) for _ in range(D // 16))

    def load_pair(ybuf, pr):
        va, vb = [], []
        for j in range(D // 16):
            w = plsc.bitcast(ybuf[pr, pl.ds(16 * j, 16)], jnp.bfloat16)
            a, b = plsc.unpack(w, format=plsc.PackFormat.INTERLEAVED)
            va.append(a)
            vb.append(b)
        return tuple(va), tuple(vb)

    def process_chunk(ci, ybuf, bbuf, carry):
        def group_body(g, carry):
            base, first, cur, cnt, accs = carry
            segv = bbuf[pl.ds(g * 16, 16)]
            allsame = jnp.all(segv == cur)

            def fast(c):
                base, first, cur, cnt, accs = c
                for r in range(8):
                    pa, pb = load_pair(ybuf, g * 8 + r)
                    accs = tuple(accs[j] + (pa[j] + pb[j])
                                 for j in range(D // 16))
                return base, first, cur, cnt + 16.0, accs

            def slow(c):
                def step(c3, sseg, yrow):
                    def same(c4):
                        base, first, cur, cnt, accs = c4
                        na = tuple(accs[j] + yrow[j]
                                   for j in range(D // 16))
                        return base, first, cur, cnt + 1.0, na

                    def boundary(c4):
                        base, first, cur, cnt, accs = c4
                        store_run(base, cur, cnt, accs)

                        def do_flush(_):
                            flush(base, first, sseg)
                            return sseg, jnp.bool_(False)

                        base, first = lax.cond(sseg - base >= WIN, do_flush,
                                               lambda _: (base, first), None)
                        return base, first, sseg, 1.0, yrow

                    return lax.cond(sseg == c3[2], same, boundary, c3)

                def pr_body(r, c2):
                    pa, pb = load_pair(ybuf, g * 8 + r)
                    c2 = step(c2, _lane(segv, 2 * r), pa)
                    c2 = step(c2, _lane(segv, 2 * r + 1), pb)
                    return c2

                return lax.fori_loop(0, 8, pr_body, c)

            return lax.cond(allsame, fast, slow, carry)

        return lax.fori_loop(0, CHUNK // 16, group_body, carry)

    def copies(ci, k, sem):
        row0 = i0 + ci * CHUNK
        prow0 = row0 // 2
        return (
            pltpu.make_async_copy(y_hbm.at[pl.ds(prow0, CHUNK // 2)],
                                  ybuf2.at[k], sem),
            pltpu.make_async_copy(batch_hbm.at[pl.ds(row0, CHUNK)],
                                  bbuf2.at[k], sem),
        )

    def start(ci, k, sem):
        for c in copies(ci, k, sem):
            c.start()

    def wait(ci, k, sem):
        for c in copies(ci, k, sem):
            c.wait()

    init = (prev_seg, jnp.bool_(True), prev_seg, jnp.float32(0.0), zaccs)

    pltpu.sync_copy(y_hbm.at[pl.ds(i0 // 2, CHUNK // 2)], ybuf2.at[0])
    pltpu.sync_copy(batch_hbm.at[pl.ds(i0, CHUNK)], bbuf2.at[0])
    carry = process_chunk(0, ybuf2.at[0], bbuf2.at[0], init)
    start(1, 0, sem0)
    start(2, 1, sem1)

    def pair_body(k, carry):
        ci0 = 1 + 2 * k
        wait(ci0, 0, sem0)
        carry = process_chunk(ci0, ybuf2.at[0], bbuf2.at[0], carry)

        @pl.when(ci0 + 2 < NCHUNK)
        def _():
            start(ci0 + 2, 0, sem0)

        wait(ci0 + 1, 1, sem1)
        carry = process_chunk(ci0 + 1, ybuf2.at[1], bbuf2.at[1], carry)

        @pl.when(ci0 + 3 < NCHUNK)
        def _():
            start(ci0 + 3, 1, sem1)

        return carry

    base, first, cur, cnt, accs = lax.fori_loop(0, (NCHUNK - 1) // 2,
                                                pair_body, carry)
    store_run(base, cur, cnt, accs)


    @pl.when(first)
    def _():
        pltpu.sync_copy(acc.at[pl.ds(0, 1)], outp_hbm.at[pl.ds(wid, 1)])

    start_r = jnp.where(first, 1, 0)

    def frow(k, _):
        r = start_r + k
        pltpu.sync_copy(acc.at[pl.ds(r, 1)], outs_hbm.at[pl.ds(base + r, 1)])
        return 0
    lax.fori_loop(0, last_seg - base - start_r + 1, frow, 0)

    @pl.when(wid == NW - 1)
    def _():
        zero_acc()

        def ztail(k, _):
            pltpu.sync_copy(acc.at[pl.ds(0, 1)],
                            outs_hbm.at[pl.ds(last_seg + 1 + k, 1)])
            return 0
        lax.fori_loop(0, B - 1 - last_seg, ztail, 0)


def _phase2(y, batch):
    mesh = plsc.VectorSubcoreMesh(core_axis_name="c", subcore_axis_name="s",
                                  num_cores=NC, num_subcores=NS)
    f = pl.kernel(
        _sc_body,
        out_type=(
            jax.ShapeDtypeStruct((B, AC), jnp.float32),
            jax.ShapeDtypeStruct((NW, AC), jnp.float32),
        ),
        mesh=mesh,
        compiler_params=pltpu.CompilerParams(use_tc_tiling_on_sc=False,
                                             needs_layout_passes=False),
        scratch_types=[
            pltpu.VMEM((2, CHUNK // 2, D), jnp.int32),
            pltpu.VMEM((2, CHUNK), jnp.int32),
            pltpu.VMEM((16,), jnp.int32),
            pltpu.VMEM((WIN, AC), jnp.float32),
            pltpu.SemaphoreType.DMA,
            pltpu.SemaphoreType.DMA,
        ],
    )
    return f(y, batch)


def _fin_body(sc_ref, s_ref, p_ref, w_ref, b_ref, o_ref):
    pinv = sc_ref[0, 0]
    q = sc_ref[0, 1]
    s = s_ref[...]
    part = p_ref[...]
    ids = part[:, AC - 16].astype(jnp.int32)
    rows = lax.broadcasted_iota(jnp.int32, (B, NW), 0)
    onehot = (rows == ids[None, :]).astype(jnp.float32)
    s = s + jnp.dot(onehot, part, preferred_element_type=jnp.float32)
    pooled = s[:, :D]
    nums = jnp.sum(s[:, D:D + 16], axis=1, keepdims=True)
    gnp = jnp.exp(pinv * jnp.log(pooled)) * jnp.exp(-q * jnp.log(nums))
    o_ref[...] = (
        jax.lax.dot_general(gnp, w_ref[...], (((1,), (1,)), ((), ())),
                            preferred_element_type=jnp.float32)
        + b_ref[...][None, :]
    )


def _phase3(scal, s, partials, W, b):
    return pl.pallas_call(
        _fin_body,
        in_specs=[
            pl.BlockSpec(memory_space=pltpu.SMEM),
            pl.BlockSpec(memory_space=pltpu.VMEM),
            pl.BlockSpec(memory_space=pltpu.VMEM),
            pl.BlockSpec(memory_space=pltpu.VMEM),
            pl.BlockSpec(memory_space=pltpu.VMEM),
        ],
        out_shape=jax.ShapeDtypeStruct((B, D), jnp.float32),
    )(scal, s, partials, W, b)


def kernel(x, batch, ps, qs, W, b):
    neg_sp = -jax.nn.softplus(ps)
    thres = jnp.where(neg_sp > -50.0, neg_sp, 50.0)
    p = -thres[0]
    q = jnp.tanh(qs)[0]
    batch = batch.astype(jnp.int32)

    y = _phase1(x, p.reshape(1, 1))
    s, partials = _phase2(y, batch)
    scal = jnp.stack([1.0 / p, q]).reshape(1, 2)
    return _phase3(scal, s, partials, W, b)


# speedup vs baseline: 7.0561x; 1.1924x over previous
import functools

import jax
import jax.numpy as jnp
from jax import lax
from jax.experimental import pallas as pl
from jax.experimental.pallas import tpu as pltpu
from jax.experimental.pallas import tpu_sc as plsc

N = 320000
D = 128
B = 1024
EPS = 1e-6

NC = 2
NS = 16
NW = NC * NS
ROWS_PER = N // NW
CHUNK = 400
NCHUNK = ROWS_PER // CHUNK
WIN = 128
AC = D + 32


def _lane(v, i):
    return jnp.sum(jnp.where(lax.iota(jnp.int32, 16) == i, v, jnp.zeros_like(v)))


_P1_BLOCK = 4000


def _pow_body(p_ref, x_ref, y_ref):
    p = p_ref[0, 0]
    xa = jnp.abs(x_ref[...]) + EPS
    yb = jnp.exp(p * jnp.log(xa)).astype(jnp.bfloat16)
    y_ref[...] = pltpu.bitcast(yb, jnp.int32)


def _phase1(x, p_arr):
    return pl.pallas_call(
        _pow_body,
        grid=(N // _P1_BLOCK,),
        in_specs=[
            pl.BlockSpec(memory_space=pltpu.SMEM),
            pl.BlockSpec((_P1_BLOCK, D), lambda i: (i, 0)),
        ],
        out_specs=pl.BlockSpec((_P1_BLOCK // 2, D), lambda i: (i, 0)),
        out_shape=jax.ShapeDtypeStruct((N // 2, D), jnp.int32),
    )(p_arr, x)


def _sc_body(y_hbm, batch_hbm, outs_hbm, outp_hbm, ybuf2, bbuf2, tmpv, acc,
             sem0, sem1):
    cid = lax.axis_index("c")
    sid = lax.axis_index("s")
    wid = sid * NC + cid
    i0 = wid * ROWS_PER

    zvec = jnp.zeros((16,), jnp.float32)

    def zero_acc():
        def zrow(r, _):
            for j in range(AC // 16):
                acc[r, pl.ds(16 * j, 16)] = zvec
            return 0
        lax.fori_loop(0, WIN, zrow, 0)

    zero_acc()

    @pl.when(wid > 0)
    def _():
        pltpu.sync_copy(batch_hbm.at[pl.ds(i0 - 16, 16)], tmpv)
    prev_raw = _lane(tmpv[...], 15)
    prev_seg = jnp.where(wid > 0, prev_raw, jnp.int32(-1))

    pltpu.sync_copy(batch_hbm.at[pl.ds(i0 + ROWS_PER - 16, 16)], tmpv)
    last_seg = _lane(tmpv[...], 15)

    idv = jnp.where(lax.iota(jnp.int32, 16) == 0,
                    prev_seg.astype(jnp.float32), 0.0)
    plsc.addupdate(acc.at[0, pl.ds(AC - 16, 16)], idv)

    def flush(base, first, new_base):
        @pl.when(first)
        def _():
            pltpu.sync_copy(acc.at[pl.ds(0, 1)], outp_hbm.at[pl.ds(wid, 1)])
            pltpu.sync_copy(acc.at[pl.ds(1, WIN - 1)],
                            outs_hbm.at[pl.ds(base + 1, WIN - 1)])

        @pl.when(jnp.logical_not(first))
        def _():
            pltpu.sync_copy(acc.at[pl.ds(0, WIN)],
                            outs_hbm.at[pl.ds(base, WIN)])

        zero_acc()

        def zfill(k, _):
            pltpu.sync_copy(acc.at[pl.ds(0, 1)],
                            outs_hbm.at[pl.ds(base + WIN + k, 1)])
            return 0
        lax.fori_loop(0, new_base - (base + WIN), zfill, 0)

    def store_run(base, cur, cnt, accs):
        row = cur - base
        for j in range(D // 16):
            acc[row, pl.ds(16 * j, 16)] = accs[j]
        acc[row, pl.ds(D, 16)] = jnp.where(lax.iota(jnp.int32, 16) == 0,
                                           cnt, 0.0)

    zaccs = tuple(jnp.zeros((16,), jnp.float32) for _ in range(D // 16))

    def load_pair(ybuf, pr):
        va, vb = [], []
        for j in range(D // 16):
            w = plsc.bitcast(ybuf[pr, pl.ds(16 * j, 16)], jnp.bfloat16)
            a, b = plsc.unpack(w, format=plsc.PackFormat.INTERLEAVED)
            va.append(a)
            vb.append(b)
        return tuple(va), tuple(vb)

    def process_chunk(ci, ybuf, bbuf, carry):
        def group_body(g, carry):
            base, first, cur, cnt, accs = carry
            segv = bbuf[pl.ds(g * 16, 16)]
            allsame = jnp.all(segv == cur)

            def fast(c):
                base, first, cur, cnt, accs = c
                for r in range(8):
                    pa, pb = load_pair(ybuf, g * 8 + r)
                    accs = tuple(accs[j] + (pa[j] + pb[j])
                                 for j in range(D // 16))
                return base, first, cur, cnt + 16.0, accs

            def slow(c):
                def step(c3, sseg, yrow):
                    def same(c4):
                        base, first, cur, cnt, accs = c4
                        na = tuple(accs[j] + yrow[j]
                                   for j in range(D // 16))
                        return base, first, cur, cnt + 1.0, na

                    def boundary(c4):
                        base, first, cur, cnt, accs = c4
                        store_run(base, cur, cnt, accs)

                        def do_flush(_):
                            flush(base, first, sseg)
                            return sseg, jnp.bool_(False)

                        base, first = lax.cond(sseg - base >= WIN, do_flush,
                                               lambda _: (base, first), None)
                        return base, first, sseg, 1.0, yrow

                    return lax.cond(sseg == c3[2], same, boundary, c3)

                def pr_body(r, c2):
                    pa, pb = load_pair(ybuf, g * 8 + r)
                    c2 = step(c2, _lane(segv, 2 * r), pa)
                    c2 = step(c2, _lane(segv, 2 * r + 1), pb)
                    return c2

                return lax.fori_loop(0, 8, pr_body, c)

            return lax.cond(allsame, fast, slow, carry)

        return lax.fori_loop(0, CHUNK // 16, group_body, carry)

    def copies(ci, k, sem):
        row0 = i0 + ci * CHUNK
        prow0 = row0 // 2
        return (
            pltpu.make_async_copy(y_hbm.at[pl.ds(prow0, CHUNK // 2)],
                                  ybuf2.at[k], sem),
            pltpu.make_async_copy(batch_hbm.at[pl.ds(row0, CHUNK)],
                                  bbuf2.at[k], sem),
        )

    def start(ci, k, sem):
        for c in copies(ci, k, sem):
            c.start()

    def wait(ci, k, sem):
        for c in copies(ci, k, sem):
            c.wait()

    init = (prev_seg, jnp.bool_(True), prev_seg, jnp.float32(0.0), zaccs)

    pltpu.sync_copy(y_hbm.at[pl.ds(i0 // 2, CHUNK // 2)], ybuf2.at[0])
    pltpu.sync_copy(batch_hbm.at[pl.ds(i0, CHUNK)], bbuf2.at[0])
    carry = process_chunk(0, ybuf2.at[0], bbuf2.at[0], init)
    start(1, 0, sem0)
    start(2, 1, sem1)

    def pair_body(k, carry):
        ci0 = 1 + 2 * k
        wait(ci0, 0, sem0)
        carry = process_chunk(ci0, ybuf2.at[0], bbuf2.at[0], carry)

        @pl.when(ci0 + 2 < NCHUNK)
        def _():
            start(ci0 + 2, 0, sem0)

        wait(ci0 + 1, 1, sem1)
        carry = process_chunk(ci0 + 1, ybuf2.at[1], bbuf2.at[1], carry)

        @pl.when(ci0 + 3 < NCHUNK)
        def _():
            start(ci0 + 3, 1, sem1)

        return carry

    base, first, cur, cnt, accs = lax.fori_loop(0, (NCHUNK - 1) // 2,
                                                pair_body, carry)
    store_run(base, cur, cnt, accs)


    @pl.when(first)
    def _():
        pltpu.sync_copy(acc.at[pl.ds(0, 1)], outp_hbm.at[pl.ds(wid, 1)])

    start_r = jnp.where(first, 1, 0)

    def frow(k, _):
        r = start_r + k
        pltpu.sync_copy(acc.at[pl.ds(r, 1)], outs_hbm.at[pl.ds(base + r, 1)])
        return 0
    lax.fori_loop(0, last_seg - base - start_r + 1, frow, 0)

    @pl.when(wid == NW - 1)
    def _():
        zero_acc()

        def ztail(k, _):
            pltpu.sync_copy(acc.at[pl.ds(0, 1)],
                            outs_hbm.at[pl.ds(last_seg + 1 + k, 1)])
            return 0
        lax.fori_loop(0, B - 1 - last_seg, ztail, 0)


def _phase2(y, batch):
    mesh = plsc.VectorSubcoreMesh(core_axis_name="c", subcore_axis_name="s",
                                  num_cores=NC, num_subcores=NS)
    f = pl.kernel(
        _sc_body,
        out_type=(
            jax.ShapeDtypeStruct((B, AC), jnp.float32),
            jax.ShapeDtypeStruct((NW, AC), jnp.float32),
        ),
        mesh=mesh,
        compiler_params=pltpu.CompilerParams(use_tc_tiling_on_sc=False,
                                             needs_layout_passes=False),
        scratch_types=[
            pltpu.VMEM((2, CHUNK // 2, D), jnp.int32),
            pltpu.VMEM((2, CHUNK), jnp.int32),
            pltpu.VMEM((16,), jnp.int32),
            pltpu.VMEM((WIN, AC), jnp.float32),
            pltpu.SemaphoreType.DMA,
            pltpu.SemaphoreType.DMA,
        ],
    )
    return f(y, batch)


def _fin_body(sc_ref, s_ref, p_ref, w_ref, b_ref, o_ref):
    pinv = sc_ref[0, 0]
    q = sc_ref[0, 1]
    s = s_ref[...]
    part = p_ref[...]
    ids = part[:, AC - 16].astype(jnp.int32)
    rows = lax.broadcasted_iota(jnp.int32, (B, NW), 0)
    onehot = (rows == ids[None, :]).astype(jnp.float32)
    s = s + jnp.dot(onehot, part, preferred_element_type=jnp.float32)
    pooled = s[:, :D]
    nums = jnp.sum(s[:, D:D + 16], axis=1, keepdims=True)
    gnp = jnp.exp(pinv * jnp.log(pooled)) * jnp.exp(-q * jnp.log(nums))
    o_ref[...] = (
        jax.lax.dot_general(gnp, w_ref[...], (((1,), (1,)), ((), ())),
                            preferred_element_type=jnp.float32)
        + b_ref[...][None, :]
    )


def _phase3(scal, s, partials, W, b):
    return pl.pallas_call(
        _fin_body,
        in_specs=[
            pl.BlockSpec(memory_space=pltpu.SMEM),
            pl.BlockSpec(memory_space=pltpu.VMEM),
            pl.BlockSpec(memory_space=pltpu.VMEM),
            pl.BlockSpec(memory_space=pltpu.VMEM),
            pl.BlockSpec(memory_space=pltpu.VMEM),
        ],
        out_shape=jax.ShapeDtypeStruct((B, D), jnp.float32),
    )(scal, s, partials, W, b)


def kernel(x, batch, ps, qs, W, b):
    neg_sp = -jax.nn.softplus(ps)
    thres = jnp.where(neg_sp > -50.0, neg_sp, 50.0)
    p = -thres[0]
    q = jnp.tanh(qs)[0]
    batch = batch.astype(jnp.int32)

    y = _phase1(x, p.reshape(1, 1))
    s, partials = _phase2(y, batch)
    scal = jnp.stack([1.0 / p, q]).reshape(1, 2)
    return _phase3(scal, s, partials, W, b)


# speedup vs baseline: 7.8981x; 1.1193x over previous
import functools

import jax
import jax.numpy as jnp
from jax import lax
from jax.experimental import pallas as pl
from jax.experimental.pallas import tpu as pltpu
from jax.experimental.pallas import tpu_sc as plsc

N = 320000
D = 128
B = 1024
EPS = 1e-6

NC = 2
NS = 16
NW = NC * NS
ROWS_PER = N // NW
CHUNK = 400
NCHUNK = ROWS_PER // CHUNK
WIN = 128
AC = D + 32


def _lane(v, i):
    return jnp.sum(jnp.where(lax.iota(jnp.int32, 16) == i, v, jnp.zeros_like(v)))


_P1_BLOCK = 8000


def _pow_body(p_ref, x_ref, y_ref):
    p = p_ref[0, 0]
    xa = jnp.abs(x_ref[...]) + EPS
    yb = jnp.exp(p * jnp.log(xa)).astype(jnp.bfloat16)
    y_ref[...] = pltpu.bitcast(yb, jnp.int32)


def _phase1(x, p_arr):
    return pl.pallas_call(
        _pow_body,
        grid=(N // _P1_BLOCK,),
        in_specs=[
            pl.BlockSpec(memory_space=pltpu.SMEM),
            pl.BlockSpec((_P1_BLOCK, D), lambda i: (i, 0)),
        ],
        out_specs=pl.BlockSpec((_P1_BLOCK // 2, D), lambda i: (i, 0)),
        out_shape=jax.ShapeDtypeStruct((N // 2, D), jnp.int32),
    )(p_arr, x)


def _sc_body(y_hbm, batch_hbm, outs_hbm, outp_hbm, ybuf2, bbuf2, tmpv, acc,
             sem0, sem1):
    cid = lax.axis_index("c")
    sid = lax.axis_index("s")
    wid = sid * NC + cid
    i0 = wid * ROWS_PER

    zvec = jnp.zeros((16,), jnp.float32)

    def zero_acc():
        def zrow(r, _):
            for j in range(AC // 16):
                acc[r, pl.ds(16 * j, 16)] = zvec
            return 0
        lax.fori_loop(0, WIN, zrow, 0)

    zero_acc()

    @pl.when(wid > 0)
    def _():
        pltpu.sync_copy(batch_hbm.at[pl.ds(i0 - 16, 16)], tmpv)
    prev_raw = _lane(tmpv[...], 15)
    prev_seg = jnp.where(wid > 0, prev_raw, jnp.int32(-1))

    pltpu.sync_copy(batch_hbm.at[pl.ds(i0 + ROWS_PER - 16, 16)], tmpv)
    last_seg = _lane(tmpv[...], 15)

    idv = jnp.where(lax.iota(jnp.int32, 16) == 0,
                    prev_seg.astype(jnp.float32), 0.0)
    plsc.addupdate(acc.at[0, pl.ds(AC - 16, 16)], idv)

    def flush(base, first, new_base):
        @pl.when(first)
        def _():
            pltpu.sync_copy(acc.at[pl.ds(0, 1)], outp_hbm.at[pl.ds(wid, 1)])
            pltpu.sync_copy(acc.at[pl.ds(1, WIN - 1)],
                            outs_hbm.at[pl.ds(base + 1, WIN - 1)])

        @pl.when(jnp.logical_not(first))
        def _():
            pltpu.sync_copy(acc.at[pl.ds(0, WIN)],
                            outs_hbm.at[pl.ds(base, WIN)])

        zero_acc()

        def zfill(k, _):
            pltpu.sync_copy(acc.at[pl.ds(0, 1)],
                            outs_hbm.at[pl.ds(base + WIN + k, 1)])
            return 0
        lax.fori_loop(0, new_base - (base + WIN), zfill, 0)

    def store_run(base, cur, cnt, accs):
        row = cur - base
        for j in range(D // 16):
            acc[row, pl.ds(16 * j, 16)] = accs[j]
        acc[row, pl.ds(D, 16)] = jnp.where(lax.iota(jnp.int32, 16) == 0,
                                           cnt, 0.0)

    zaccs = tuple(jnp.zeros((16,), jnp.float32) for _ in range(D // 16))

    def load_pair(ybuf, pr):
        va, vb = [], []
        for j in range(D // 16):
            w = plsc.bitcast(ybuf[pr, pl.ds(16 * j, 16)], jnp.bfloat16)
            a, b = plsc.unpack(w, format=plsc.PackFormat.INTERLEAVED)
            va.append(a)
            vb.append(b)
        return tuple(va), tuple(vb)

    def process_chunk(ci, ybuf, bbuf, carry):
        def group_body(g, carry):
            base, first, cur, cnt, accs = carry
            segv = bbuf[pl.ds(g * 16, 16)]
            allsame = jnp.all(segv == cur)

            def fast(c):
                base, first, cur, cnt, accs = c
                for r in range(8):
                    pa, pb = load_pair(ybuf, g * 8 + r)
                    accs = tuple(accs[j] + (pa[j] + pb[j])
                                 for j in range(D // 16))
                return base, first, cur, cnt + 16.0, accs

            def slow(c):
                def step(c3, sseg, yrow):
                    def same(c4):
                        base, first, cur, cnt, accs = c4
                        na = tuple(accs[j] + yrow[j]
                                   for j in range(D // 16))
                        return base, first, cur, cnt + 1.0, na

                    def boundary(c4):
                        base, first, cur, cnt, accs = c4
                        store_run(base, cur, cnt, accs)

                        def do_flush(_):
                            flush(base, first, sseg)
                            return sseg, jnp.bool_(False)

                        base, first = lax.cond(sseg - base >= WIN, do_flush,
                                               lambda _: (base, first), None)
                        return base, first, sseg, 1.0, yrow

                    return lax.cond(sseg == c3[2], same, boundary, c3)

                def pr_body(r, c2):
                    pa, pb = load_pair(ybuf, g * 8 + r)
                    c2 = step(c2, _lane(segv, 2 * r), pa)
                    c2 = step(c2, _lane(segv, 2 * r + 1), pb)
                    return c2

                return lax.fori_loop(0, 8, pr_body, c)

            return lax.cond(allsame, fast, slow, carry)

        return lax.fori_loop(0, CHUNK // 16, group_body, carry)

    def copies(ci, k, sem):
        row0 = i0 + ci * CHUNK
        prow0 = row0 // 2
        return (
            pltpu.make_async_copy(y_hbm.at[pl.ds(prow0, CHUNK // 2)],
                                  ybuf2.at[k], sem),
            pltpu.make_async_copy(batch_hbm.at[pl.ds(row0, CHUNK)],
                                  bbuf2.at[k], sem),
        )

    def start(ci, k, sem):
        for c in copies(ci, k, sem):
            c.start()

    def wait(ci, k, sem):
        for c in copies(ci, k, sem):
            c.wait()

    init = (prev_seg, jnp.bool_(True), prev_seg, jnp.float32(0.0), zaccs)

    pltpu.sync_copy(y_hbm.at[pl.ds(i0 // 2, CHUNK // 2)], ybuf2.at[0])
    pltpu.sync_copy(batch_hbm.at[pl.ds(i0, CHUNK)], bbuf2.at[0])
    carry = process_chunk(0, ybuf2.at[0], bbuf2.at[0], init)
    start(1, 0, sem0)
    start(2, 1, sem1)

    def pair_body(k, carry):
        ci0 = 1 + 2 * k
        wait(ci0, 0, sem0)
        carry = process_chunk(ci0, ybuf2.at[0], bbuf2.at[0], carry)

        @pl.when(ci0 + 2 < NCHUNK)
        def _():
            start(ci0 + 2, 0, sem0)

        wait(ci0 + 1, 1, sem1)
        carry = process_chunk(ci0 + 1, ybuf2.at[1], bbuf2.at[1], carry)

        @pl.when(ci0 + 3 < NCHUNK)
        def _():
            start(ci0 + 3, 1, sem1)

        return carry

    base, first, cur, cnt, accs = lax.fori_loop(0, (NCHUNK - 1) // 2,
                                                pair_body, carry)
    store_run(base, cur, cnt, accs)


    @pl.when(first)
    def _():
        pltpu.sync_copy(acc.at[pl.ds(0, 1)], outp_hbm.at[pl.ds(wid, 1)])

    start_r = jnp.where(first, 1, 0)

    def frow(k, _):
        r = start_r + k
        pltpu.sync_copy(acc.at[pl.ds(r, 1)], outs_hbm.at[pl.ds(base + r, 1)])
        return 0
    lax.fori_loop(0, last_seg - base - start_r + 1, frow, 0)

    @pl.when(wid == NW - 1)
    def _():
        zero_acc()

        def ztail(k, _):
            pltpu.sync_copy(acc.at[pl.ds(0, 1)],
                            outs_hbm.at[pl.ds(last_seg + 1 + k, 1)])
            return 0
        lax.fori_loop(0, B - 1 - last_seg, ztail, 0)


def _phase2(y, batch):
    mesh = plsc.VectorSubcoreMesh(core_axis_name="c", subcore_axis_name="s",
                                  num_cores=NC, num_subcores=NS)
    f = pl.kernel(
        _sc_body,
        out_type=(
            jax.ShapeDtypeStruct((B, AC), jnp.float32),
            jax.ShapeDtypeStruct((NW, AC), jnp.float32),
        ),
        mesh=mesh,
        compiler_params=pltpu.CompilerParams(use_tc_tiling_on_sc=False,
                                             needs_layout_passes=False),
        scratch_types=[
            pltpu.VMEM((2, CHUNK // 2, D), jnp.int32),
            pltpu.VMEM((2, CHUNK), jnp.int32),
            pltpu.VMEM((16,), jnp.int32),
            pltpu.VMEM((WIN, AC), jnp.float32),
            pltpu.SemaphoreType.DMA,
            pltpu.SemaphoreType.DMA,
        ],
    )
    return f(y, batch)


def _fin_body(sc_ref, s_ref, p_ref, w_ref, b_ref, o_ref):
    pinv = sc_ref[0, 0]
    q = sc_ref[0, 1]
    s = s_ref[...]
    part = p_ref[...]
    ids = part[:, AC - 16].astype(jnp.int32)
    rows = lax.broadcasted_iota(jnp.int32, (B, NW), 0)
    onehot = (rows == ids[None, :]).astype(jnp.float32)
    s = s + jnp.dot(onehot, part, preferred_element_type=jnp.float32)
    pooled = s[:, :D]
    nums = jnp.sum(s[:, D:D + 16], axis=1, keepdims=True)
    gnp = jnp.exp(pinv * jnp.log(pooled)) * jnp.exp(-q * jnp.log(nums))
    o_ref[...] = (
        jax.lax.dot_general(gnp, w_ref[...], (((1,), (1,)), ((), ())),
                            preferred_element_type=jnp.float32)
        + b_ref[...][None, :]
    )


def _phase3(scal, s, partials, W, b):
    return pl.pallas_call(
        _fin_body,
        in_specs=[
            pl.BlockSpec(memory_space=pltpu.SMEM),
            pl.BlockSpec(memory_space=pltpu.VMEM),
            pl.BlockSpec(memory_space=pltpu.VMEM),
            pl.BlockSpec(memory_space=pltpu.VMEM),
            pl.BlockSpec(memory_space=pltpu.VMEM),
        ],
        out_shape=jax.ShapeDtypeStruct((B, D), jnp.float32),
    )(scal, s, partials, W, b)


def kernel(x, batch, ps, qs, W, b):
    neg_sp = -jax.nn.softplus(ps)
    thres = jnp.where(neg_sp > -50.0, neg_sp, 50.0)
    p = -thres[0]
    q = jnp.tanh(qs)[0]
    batch = batch.astype(jnp.int32)

    y = _phase1(x, p.reshape(1, 1))
    s, partials = _phase2(y, batch)
    scal = jnp.stack([1.0 / p, q]).reshape(1, 2)
    return _phase3(scal, s, partials, W, b)


# speedup vs baseline: 8.0885x; 1.0241x over previous
import functools

import jax
import jax.numpy as jnp
from jax import lax
from jax.experimental import pallas as pl
from jax.experimental.pallas import tpu as pltpu
from jax.experimental.pallas import tpu_sc as plsc

N = 320000
D = 128
B = 1024
EPS = 1e-6

NC = 2
NS = 16
NW = NC * NS
ROWS_PER = N // NW
CHUNK = 400
NCHUNK = ROWS_PER // CHUNK
WIN = 128
AC = D + 32


def _lane(v, i):
    return jnp.sum(jnp.where(lax.iota(jnp.int32, 16) == i, v, jnp.zeros_like(v)))


_P1_BLOCK = 16000


def _pow_body(p_ref, x_ref, y_ref):
    p = p_ref[0, 0]
    xa = jnp.abs(x_ref[...]) + EPS
    yb = jnp.exp(p * jnp.log(xa)).astype(jnp.bfloat16)
    y_ref[...] = pltpu.bitcast(yb, jnp.int32)


def _phase1(x, p_arr):
    return pl.pallas_call(
        _pow_body,
        grid=(N // _P1_BLOCK,),
        in_specs=[
            pl.BlockSpec(memory_space=pltpu.SMEM),
            pl.BlockSpec((_P1_BLOCK, D), lambda i: (i, 0)),
        ],
        out_specs=pl.BlockSpec((_P1_BLOCK // 2, D), lambda i: (i, 0)),
        out_shape=jax.ShapeDtypeStruct((N // 2, D), jnp.int32),
    )(p_arr, x)


def _sc_body(y_hbm, batch_hbm, outs_hbm, outp_hbm, ybuf2, bbuf2, tmpv, acc,
             sem0, sem1):
    cid = lax.axis_index("c")
    sid = lax.axis_index("s")
    wid = sid * NC + cid
    i0 = wid * ROWS_PER

    zvec = jnp.zeros((16,), jnp.float32)

    def zero_acc():
        def zrow(r, _):
            for j in range(AC // 16):
                acc[r, pl.ds(16 * j, 16)] = zvec
            return 0
        lax.fori_loop(0, WIN, zrow, 0)

    zero_acc()

    @pl.when(wid > 0)
    def _():
        pltpu.sync_copy(batch_hbm.at[pl.ds(i0 - 16, 16)], tmpv)
    prev_raw = _lane(tmpv[...], 15)
    prev_seg = jnp.where(wid > 0, prev_raw, jnp.int32(-1))

    pltpu.sync_copy(batch_hbm.at[pl.ds(i0 + ROWS_PER - 16, 16)], tmpv)
    last_seg = _lane(tmpv[...], 15)

    idv = jnp.where(lax.iota(jnp.int32, 16) == 0,
                    prev_seg.astype(jnp.float32), 0.0)
    plsc.addupdate(acc.at[0, pl.ds(AC - 16, 16)], idv)

    def flush(base, first, new_base):
        @pl.when(first)
        def _():
            pltpu.sync_copy(acc.at[pl.ds(0, 1)], outp_hbm.at[pl.ds(wid, 1)])
            pltpu.sync_copy(acc.at[pl.ds(1, WIN - 1)],
                            outs_hbm.at[pl.ds(base + 1, WIN - 1)])

        @pl.when(jnp.logical_not(first))
        def _():
            pltpu.sync_copy(acc.at[pl.ds(0, WIN)],
                            outs_hbm.at[pl.ds(base, WIN)])

        zero_acc()

        def zfill(k, _):
            pltpu.sync_copy(acc.at[pl.ds(0, 1)],
                            outs_hbm.at[pl.ds(base + WIN + k, 1)])
            return 0
        lax.fori_loop(0, new_base - (base + WIN), zfill, 0)

    def store_run(base, cur, cnt, accs):
        row = cur - base
        for j in range(D // 16):
            acc[row, pl.ds(16 * j, 16)] = accs[j]
        acc[row, pl.ds(D, 16)] = jnp.where(lax.iota(jnp.int32, 16) == 0,
                                           cnt, 0.0)

    zaccs = tuple(jnp.zeros((16,), jnp.float32) for _ in range(D // 16))

    def load_pair(ybuf, pr):
        va, vb = [], []
        for j in range(D // 16):
            w = plsc.bitcast(ybuf[pr, pl.ds(16 * j, 16)], jnp.bfloat16)
            a, b = plsc.unpack(w, format=plsc.PackFormat.INTERLEAVED)
            va.append(a)
            vb.append(b)
        return tuple(va), tuple(vb)

    def process_chunk(ci, ybuf, bbuf, carry):
        def group_body(g, carry):
            base, first, cur, cnt, accs = carry
            segv = bbuf[pl.ds(g * 16, 16)]
            allsame = jnp.all(segv == cur)

            def fast(c):
                base, first, cur, cnt, accs = c
                for r in range(8):
                    pa, pb = load_pair(ybuf, g * 8 + r)
                    accs = tuple(accs[j] + (pa[j] + pb[j])
                                 for j in range(D // 16))
                return base, first, cur, cnt + 16.0, accs

            def slow(c):
                def step(c3, sseg, yrow):
                    def same(c4):
                        base, first, cur, cnt, accs = c4
                        na = tuple(accs[j] + yrow[j]
                                   for j in range(D // 16))
                        return base, first, cur, cnt + 1.0, na

                    def boundary(c4):
                        base, first, cur, cnt, accs = c4
                        store_run(base, cur, cnt, accs)

                        def do_flush(_):
                            flush(base, first, sseg)
                            return sseg, jnp.bool_(False)

                        base, first = lax.cond(sseg - base >= WIN, do_flush,
                                               lambda _: (base, first), None)
                        return base, first, sseg, 1.0, yrow

                    return lax.cond(sseg == c3[2], same, boundary, c3)

                def pr_body(r, c2):
                    pa, pb = load_pair(ybuf, g * 8 + r)
                    c2 = step(c2, _lane(segv, 2 * r), pa)
                    c2 = step(c2, _lane(segv, 2 * r + 1), pb)
                    return c2

                return lax.fori_loop(0, 8, pr_body, c)

            return lax.cond(allsame, fast, slow, carry)

        return lax.fori_loop(0, CHUNK // 16, group_body, carry)

    def copies(ci, k, sem):
        row0 = i0 + ci * CHUNK
        prow0 = row0 // 2
        return (
            pltpu.make_async_copy(y_hbm.at[pl.ds(prow0, CHUNK // 2)],
                                  ybuf2.at[k], sem),
            pltpu.make_async_copy(batch_hbm.at[pl.ds(row0, CHUNK)],
                                  bbuf2.at[k], sem),
        )

    def start(ci, k, sem):
        for c in copies(ci, k, sem):
            c.start()

    def wait(ci, k, sem):
        for c in copies(ci, k, sem):
            c.wait()

    init = (prev_seg, jnp.bool_(True), prev_seg, jnp.float32(0.0), zaccs)

    pltpu.sync_copy(y_hbm.at[pl.ds(i0 // 2, CHUNK // 2)], ybuf2.at[0])
    pltpu.sync_copy(batch_hbm.at[pl.ds(i0, CHUNK)], bbuf2.at[0])
    carry = process_chunk(0, ybuf2.at[0], bbuf2.at[0], init)
    start(1, 0, sem0)
    start(2, 1, sem1)

    def pair_body(k, carry):
        ci0 = 1 + 2 * k
        wait(ci0, 0, sem0)
        carry = process_chunk(ci0, ybuf2.at[0], bbuf2.at[0], carry)

        @pl.when(ci0 + 2 < NCHUNK)
        def _():
            start(ci0 + 2, 0, sem0)

        wait(ci0 + 1, 1, sem1)
        carry = process_chunk(ci0 + 1, ybuf2.at[1], bbuf2.at[1], carry)

        @pl.when(ci0 + 3 < NCHUNK)
        def _():
            start(ci0 + 3, 1, sem1)

        return carry

    base, first, cur, cnt, accs = lax.fori_loop(0, (NCHUNK - 1) // 2,
                                                pair_body, carry)
    store_run(base, cur, cnt, accs)


    @pl.when(first)
    def _():
        pltpu.sync_copy(acc.at[pl.ds(0, 1)], outp_hbm.at[pl.ds(wid, 1)])

    start_r = jnp.where(first, 1, 0)

    def frow(k, _):
        r = start_r + k
        pltpu.sync_copy(acc.at[pl.ds(r, 1)], outs_hbm.at[pl.ds(base + r, 1)])
        return 0
    lax.fori_loop(0, last_seg - base - start_r + 1, frow, 0)

    @pl.when(wid == NW - 1)
    def _():
        zero_acc()

        def ztail(k, _):
            pltpu.sync_copy(acc.at[pl.ds(0, 1)],
                            outs_hbm.at[pl.ds(last_seg + 1 + k, 1)])
            return 0
        lax.fori_loop(0, B - 1 - last_seg, ztail, 0)


def _phase2(y, batch):
    mesh = plsc.VectorSubcoreMesh(core_axis_name="c", subcore_axis_name="s",
                                  num_cores=NC, num_subcores=NS)
    f = pl.kernel(
        _sc_body,
        out_type=(
            jax.ShapeDtypeStruct((B, AC), jnp.float32),
            jax.ShapeDtypeStruct((NW, AC), jnp.float32),
        ),
        mesh=mesh,
        compiler_params=pltpu.CompilerParams(use_tc_tiling_on_sc=False,
                                             needs_layout_passes=False),
        scratch_types=[
            pltpu.VMEM((2, CHUNK // 2, D), jnp.int32),
            pltpu.VMEM((2, CHUNK), jnp.int32),
            pltpu.VMEM((16,), jnp.int32),
            pltpu.VMEM((WIN, AC), jnp.float32),
            pltpu.SemaphoreType.DMA,
            pltpu.SemaphoreType.DMA,
        ],
    )
    return f(y, batch)


def _fin_body(sc_ref, s_ref, p_ref, w_ref, b_ref, o_ref):
    pinv = sc_ref[0, 0]
    q = sc_ref[0, 1]
    s = s_ref[...]
    part = p_ref[...]
    ids = part[:, AC - 16].astype(jnp.int32)
    rows = lax.broadcasted_iota(jnp.int32, (B, NW), 0)
    onehot = (rows == ids[None, :]).astype(jnp.float32)
    s = s + jnp.dot(onehot, part, preferred_element_type=jnp.float32)
    pooled = s[:, :D]
    nums = jnp.sum(s[:, D:D + 16], axis=1, keepdims=True)
    gnp = jnp.exp(pinv * jnp.log(pooled)) * jnp.exp(-q * jnp.log(nums))
    o_ref[...] = (
        jax.lax.dot_general(gnp, w_ref[...], (((1,), (1,)), ((), ())),
                            preferred_element_type=jnp.float32)
        + b_ref[...][None, :]
    )


def _phase3(scal, s, partials, W, b):
    return pl.pallas_call(
        _fin_body,
        in_specs=[
            pl.BlockSpec(memory_space=pltpu.SMEM),
            pl.BlockSpec(memory_space=pltpu.VMEM),
            pl.BlockSpec(memory_space=pltpu.VMEM),
            pl.BlockSpec(memory_space=pltpu.VMEM),
            pl.BlockSpec(memory_space=pltpu.VMEM),
        ],
        out_shape=jax.ShapeDtypeStruct((B, D), jnp.float32),
    )(scal, s, partials, W, b)


def kernel(x, batch, ps, qs, W, b):
    neg_sp = -jax.nn.softplus(ps)
    thres = jnp.where(neg_sp > -50.0, neg_sp, 50.0)
    p = -thres[0]
    q = jnp.tanh(qs)[0]
    batch = batch.astype(jnp.int32)

    y = _phase1(x, p.reshape(1, 1))
    s, partials = _phase2(y, batch)
    scal = jnp.stack([1.0 / p, q]).reshape(1, 2)
    return _phase3(scal, s, partials, W, b)
